# baseline (speedup 1.0000x reference)
import functools

import jax
import jax.numpy as jnp
from jax.experimental import pallas as pl
from jax.experimental.pallas import tpu as pltpu

E = 8
K = 2
D = 1024
H = 2048
N = 2048

TN = 512


def _router_kernel(x_ref, wr_ref, br_ref, mask_ref, loss_ref):
    x = x_ref[...]
    logits = jnp.dot(x, wr_ref[...], preferred_element_type=jnp.float32)
    logits = logits + br_ref[...][None, :]
    logits = logits - jnp.max(logits, axis=-1, keepdims=True)
    ex = jnp.exp(logits)
    scores = ex / jnp.sum(ex, axis=-1, keepdims=True)

    col = jax.lax.broadcasted_iota(jnp.int32, scores.shape, 1)
    v1 = jnp.max(scores, axis=-1, keepdims=True)
    is1 = scores == v1
    i1 = jnp.min(jnp.where(is1, col, E), axis=-1, keepdims=True)
    m1 = col == i1
    rest = jnp.where(m1, -jnp.inf, scores)
    v2 = jnp.max(rest, axis=-1, keepdims=True)
    is2 = rest == v2
    i2 = jnp.min(jnp.where(is2, col, E), axis=-1, keepdims=True)
    m2 = col == i2
    mask = jnp.where(m1, v1, 0.0) + jnp.where(m2, v2, 0.0)
    mask_ref[...] = mask

    importance = jnp.sum(mask, axis=0)
    imean = jnp.mean(importance)
    ivar = jnp.sum((importance - imean) ** 2) / (E - 1)
    loss_ref[...] = jnp.reshape(ivar / (imean * imean + 1e-9), (1, 1))


def _ffn_kernel(x_ref, w1_ref, b1_ref, w2_ref, b2_ref, mask_ref,
                out_ref, acc_ref):
    e = pl.program_id(0)
    n = pl.program_id(1)
    x = x_ref[...].astype(jnp.bfloat16)
    h = jnp.dot(x, w1_ref[0].astype(jnp.bfloat16),
                preferred_element_type=jnp.float32)
    h = jnp.maximum(h + b1_ref[0], 0.0).astype(jnp.bfloat16)
    o = jnp.dot(h, w2_ref[0].astype(jnp.bfloat16),
                preferred_element_type=jnp.float32)
    o = o + b2_ref[0]
    m = mask_ref[...]
    col = jax.lax.broadcasted_iota(jnp.int32, m.shape, 1)
    gate = jnp.sum(jnp.where(col == e, m, 0.0), axis=1, keepdims=True)
    contrib = o * gate

    @pl.when(e == 0)
    def _init():
        acc_ref[n] = contrib

    @pl.when(e > 0)
    def _acc():
        acc_ref[n] = acc_ref[n] + contrib

    @pl.when(e == E - 1)
    def _emit():
        out_ref[...] = acc_ref[n]


@jax.jit
def kernel(x, Wr, br, W1, b1, W2, b2):
    mask, loss = pl.pallas_call(
        _router_kernel,
        out_shape=(
            jax.ShapeDtypeStruct((N, E), jnp.float32),
            jax.ShapeDtypeStruct((1, 1), jnp.float32),
        ),
        in_specs=[
            pl.BlockSpec((N, D), lambda: (0, 0)),
            pl.BlockSpec((D, E), lambda: (0, 0)),
            pl.BlockSpec((E,), lambda: (0,)),
        ],
        out_specs=(
            pl.BlockSpec((N, E), lambda: (0, 0)),
            pl.BlockSpec((1, 1), lambda: (0, 0)),
        ),
    )(x, Wr, br)

    nt = N // TN
    out = pl.pallas_call(
        _ffn_kernel,
        grid=(E, nt),
        out_shape=jax.ShapeDtypeStruct((N, D), jnp.float32),
        in_specs=[
            pl.BlockSpec((TN, D), lambda e, n: (n, 0)),
            pl.BlockSpec((1, D, H), lambda e, n: (e, 0, 0)),
            pl.BlockSpec((1, 1, H), lambda e, n: (e, 0, 0)),
            pl.BlockSpec((1, H, D), lambda e, n: (e, 0, 0)),
            pl.BlockSpec((1, 1, D), lambda e, n: (e, 0, 0)),
            pl.BlockSpec((TN, E), lambda e, n: (n, 0)),
        ],
        out_specs=pl.BlockSpec((TN, D), lambda e, n: (n, 0)),
        scratch_shapes=[pltpu.VMEM((N // TN, TN, D), jnp.float32)],
    )(x, W1, b1.reshape(E, 1, H), W2, b2.reshape(E, 1, D), mask)

    return out, loss[0, 0]


# speedup vs baseline: 1.0076x; 1.0076x over previous
import functools

import jax
import jax.numpy as jnp
from jax.experimental import pallas as pl
from jax.experimental.pallas import tpu as pltpu

E = 8
K = 2
D = 1024
H = 2048
N = 2048

TN = 512


def _router_kernel(x_ref, wr_ref, br_ref, mask_ref, loss_ref):
    x = x_ref[...]
    logits = jnp.dot(x, wr_ref[...], preferred_element_type=jnp.float32)
    logits = logits + br_ref[...][None, :]
    logits = logits - jnp.max(logits, axis=-1, keepdims=True)
    ex = jnp.exp(logits)
    scores = ex / jnp.sum(ex, axis=-1, keepdims=True)

    col = jax.lax.broadcasted_iota(jnp.int32, scores.shape, 1)
    v1 = jnp.max(scores, axis=-1, keepdims=True)
    is1 = scores == v1
    i1 = jnp.min(jnp.where(is1, col, E), axis=-1, keepdims=True)
    m1 = col == i1
    rest = jnp.where(m1, -jnp.inf, scores)
    v2 = jnp.max(rest, axis=-1, keepdims=True)
    is2 = rest == v2
    i2 = jnp.min(jnp.where(is2, col, E), axis=-1, keepdims=True)
    m2 = col == i2
    mask = jnp.where(m1, v1, 0.0) + jnp.where(m2, v2, 0.0)
    mask_ref[...] = mask

    importance = jnp.sum(mask, axis=0)
    imean = jnp.mean(importance)
    ivar = jnp.sum((importance - imean) ** 2) / (E - 1)
    loss_ref[...] = jnp.reshape(ivar / (imean * imean + 1e-9), (1, 1))


def _ffn_kernel(x_ref, w1_ref, b1_ref, w2_ref, b2_ref, mask_ref,
                out_ref, acc_ref):
    e = pl.program_id(0)
    n = pl.program_id(1)
    x = x_ref[pl.ds(n * TN, TN), :].astype(jnp.bfloat16)
    h = jnp.dot(x, w1_ref[0].astype(jnp.bfloat16),
                preferred_element_type=jnp.float32)
    h = jnp.maximum(h + b1_ref[0], 0.0).astype(jnp.bfloat16)
    o = jnp.dot(h, w2_ref[0].astype(jnp.bfloat16),
                preferred_element_type=jnp.float32)
    o = o + b2_ref[0]
    m = mask_ref[...]
    col = jax.lax.broadcasted_iota(jnp.int32, m.shape, 1)
    gate = jnp.sum(jnp.where(col == e, m, 0.0), axis=1, keepdims=True)
    contrib = o * gate

    @pl.when(e == 0)
    def _init():
        acc_ref[n] = contrib

    @pl.when(e > 0)
    def _acc():
        acc_ref[n] = acc_ref[n] + contrib

    @pl.when(e == E - 1)
    def _emit():
        out_ref[...] = acc_ref[n]


@jax.jit
def kernel(x, Wr, br, W1, b1, W2, b2):
    mask, loss = pl.pallas_call(
        _router_kernel,
        out_shape=(
            jax.ShapeDtypeStruct((N, E), jnp.float32),
            jax.ShapeDtypeStruct((1, 1), jnp.float32),
        ),
        in_specs=[
            pl.BlockSpec((N, D), lambda: (0, 0)),
            pl.BlockSpec((D, E), lambda: (0, 0)),
            pl.BlockSpec((E,), lambda: (0,)),
        ],
        out_specs=(
            pl.BlockSpec((N, E), lambda: (0, 0)),
            pl.BlockSpec((1, 1), lambda: (0, 0)),
        ),
    )(x, Wr, br)

    nt = N // TN
    out = pl.pallas_call(
        _ffn_kernel,
        grid=(E, nt),
        out_shape=jax.ShapeDtypeStruct((N, D), jnp.float32),
        in_specs=[
            pl.BlockSpec((N, D), lambda e, n: (0, 0)),
            pl.BlockSpec((1, D, H), lambda e, n: (e, 0, 0)),
            pl.BlockSpec((1, 1, H), lambda e, n: (e, 0, 0)),
            pl.BlockSpec((1, H, D), lambda e, n: (e, 0, 0)),
            pl.BlockSpec((1, 1, D), lambda e, n: (e, 0, 0)),
            pl.BlockSpec((TN, E), lambda e, n: (n, 0)),
        ],
        out_specs=pl.BlockSpec((TN, D), lambda e, n: (n, 0)),
        scratch_shapes=[pltpu.VMEM((N // TN, TN, D), jnp.float32)],
    )(x, W1, b1.reshape(E, 1, H), W2, b2.reshape(E, 1, D), mask)

    return out, loss[0, 0]
